# baseline (speedup 1.0000x reference)
import functools

import jax
import jax.numpy as jnp
from jax import lax
from jax.experimental import pallas as pl
from jax.experimental.pallas import tpu as pltpu
from jax.experimental.pallas import tpu_sc as plsc

BATCH = 16384
EMBED_DIM = 32
HIDDEN_DIM = 64
EPS = 1e-5

LANES = 128
PACK = LANES // EMBED_DIM
TABLE_ROWS = 1000001
WIDE_ROWS = (TABLE_ROWS + PACK * 8 - 1) // (PACK * 8) * 8

NUM_CORES = 2
NUM_SUBCORES = 16
NUM_WORKERS = NUM_CORES * NUM_SUBCORES
ROWS_PER_WORKER = BATCH // NUM_WORKERS
GATHER_CHUNK = 128
CHUNKS_PER_WORKER = ROWS_PER_WORKER // GATHER_CHUNK


def _sc_gather(lin, idx2d):
    mesh = plsc.VectorSubcoreMesh(core_axis_name="c", subcore_axis_name="s")

    @functools.partial(
        pl.kernel,
        mesh=mesh,
        out_type=jax.ShapeDtypeStruct((BATCH, LANES), jnp.float32),
        scratch_types=[
            pltpu.VMEM((CHUNKS_PER_WORKER, GATHER_CHUNK), jnp.int32),
            pltpu.VMEM((ROWS_PER_WORKER, LANES), jnp.float32),
            pltpu.SemaphoreType.DMA,
        ],
    )
    def gather_kernel(lin_hbm, idx_hbm, out_hbm, idx_v, rows_v, sem):
        wid = lax.axis_index("s") * NUM_CORES + lax.axis_index("c")
        pltpu.sync_copy(idx_hbm.at[pl.ds(wid * CHUNKS_PER_WORKER, CHUNKS_PER_WORKER)], idx_v)
        copies = []
        for j in range(CHUNKS_PER_WORKER):
            copies.append(
                pltpu.async_copy(
                    lin_hbm.at[idx_v.at[j]],
                    rows_v.at[pl.ds(j * GATHER_CHUNK, GATHER_CHUNK)],
                    sem,
                )
            )
        for c in copies:
            c.wait()
        pltpu.sync_copy(rows_v, out_hbm.at[pl.ds(wid * ROWS_PER_WORKER, ROWS_PER_WORKER)])

    return gather_kernel(lin, idx2d)


_MLP_BLOCK = 2048


def _mlp_body(emb_ref, rmod_ref, w_ref, b_ref, g_ref, be_ref, out_ref):
    emb = emb_ref[...]
    group = lax.broadcasted_iota(jnp.int32, emb.shape, 1) // EMBED_DIM
    emb_sel = jnp.where(group == rmod_ref[...], emb, 0.0)
    h = lax.dot_general(
        emb_sel,
        w_ref[...],
        (((1,), (0,)), ((), ())),
        precision=lax.Precision.HIGHEST,
        preferred_element_type=jnp.float32,
    )
    h = h + b_ref[...]
    mu = jnp.mean(h, axis=1, keepdims=True)
    var = jnp.mean((h - mu) ** 2, axis=1, keepdims=True)
    hn = (h - mu) * lax.rsqrt(var + EPS)
    out_ref[...] = jnp.maximum(hn * g_ref[...] + be_ref[...], 0.0)


def _tc_mlp(emb, rmod, W_rep, b, gamma, beta):
    grid = (BATCH // _MLP_BLOCK,)
    return pl.pallas_call(
        _mlp_body,
        grid=grid,
        in_specs=[
            pl.BlockSpec((_MLP_BLOCK, LANES), lambda i: (i, 0)),
            pl.BlockSpec((_MLP_BLOCK, 1), lambda i: (i, 0)),
            pl.BlockSpec((LANES, HIDDEN_DIM), lambda i: (0, 0)),
            pl.BlockSpec((1, HIDDEN_DIM), lambda i: (0, 0)),
            pl.BlockSpec((1, HIDDEN_DIM), lambda i: (0, 0)),
            pl.BlockSpec((1, HIDDEN_DIM), lambda i: (0, 0)),
        ],
        out_specs=pl.BlockSpec((_MLP_BLOCK, HIDDEN_DIM), lambda i: (i, 0)),
        out_shape=jax.ShapeDtypeStruct((BATCH, HIDDEN_DIM), jnp.float32),
    )(emb, rmod, W_rep, b, gamma, beta)


def kernel(task_ids, table, W, b, gamma, beta):
    ids = task_ids.reshape(BATCH).astype(jnp.int32)
    lin = jnp.pad(table, ((0, WIDE_ROWS * PACK - TABLE_ROWS), (0, 0))).reshape(
        WIDE_ROWS, LANES
    )
    idx2d = (ids // PACK).reshape(BATCH // GATHER_CHUNK, GATHER_CHUNK)
    rmod = (ids % PACK).reshape(BATCH, 1)
    emb128 = _sc_gather(lin, idx2d)
    W_rep = jnp.tile(W, (PACK, 1))
    return _tc_mlp(
        emb128,
        rmod,
        W_rep,
        b.reshape(1, HIDDEN_DIM),
        gamma.reshape(1, HIDDEN_DIM),
        beta.reshape(1, HIDDEN_DIM),
    )


# speedup vs baseline: 1.4761x; 1.4761x over previous
import functools

import jax
import jax.numpy as jnp
from jax import lax
from jax.experimental import pallas as pl
from jax.experimental.pallas import tpu as pltpu
from jax.experimental.pallas import tpu_sc as plsc

BATCH = 16384
EMBED_DIM = 32
HIDDEN_DIM = 64
EPS = 1e-5

LANES = 128
PACK = LANES // EMBED_DIM
TABLE_ROWS = 1000001
_DETILE_BLOCK = 2048
WIDE_ROWS = 123 * _DETILE_BLOCK

NUM_CORES = 2
NUM_SUBCORES = 16
NUM_WORKERS = NUM_CORES * NUM_SUBCORES
ROWS_PER_WORKER = BATCH // NUM_WORKERS
GATHER_CHUNK = 128
CHUNKS_PER_WORKER = ROWS_PER_WORKER // GATHER_CHUNK


def _sc_gather(lin, idx2d):
    mesh = plsc.VectorSubcoreMesh(core_axis_name="c", subcore_axis_name="s")

    @functools.partial(
        pl.kernel,
        mesh=mesh,
        out_type=jax.ShapeDtypeStruct((BATCH, LANES), jnp.float32),
        scratch_types=[
            pltpu.VMEM((CHUNKS_PER_WORKER, GATHER_CHUNK), jnp.int32),
            pltpu.VMEM((ROWS_PER_WORKER, LANES), jnp.float32),
            pltpu.SemaphoreType.DMA,
        ],
    )
    def gather_kernel(lin_hbm, idx_hbm, out_hbm, idx_v, rows_v, sem):
        wid = lax.axis_index("s") * NUM_CORES + lax.axis_index("c")
        pltpu.sync_copy(idx_hbm.at[pl.ds(wid * CHUNKS_PER_WORKER, CHUNKS_PER_WORKER)], idx_v)
        copies = []
        for j in range(CHUNKS_PER_WORKER):
            copies.append(
                pltpu.async_copy(
                    lin_hbm.at[idx_v.at[j]],
                    rows_v.at[pl.ds(j * GATHER_CHUNK, GATHER_CHUNK)],
                    sem,
                )
            )
        for c in copies:
            c.wait()
        pltpu.sync_copy(rows_v, out_hbm.at[pl.ds(wid * ROWS_PER_WORKER, ROWS_PER_WORKER)])

    return gather_kernel(lin, idx2d)


def _detile_body(in0, in1, in2, in3, out_ref):
    out_ref[:, 0:EMBED_DIM] = in0[...]
    out_ref[:, EMBED_DIM : 2 * EMBED_DIM] = in1[...]
    out_ref[:, 2 * EMBED_DIM : 3 * EMBED_DIM] = in2[...]
    out_ref[:, 3 * EMBED_DIM :] = in3[...]


def _tc_detile(table):
    nblk = WIDE_ROWS // _DETILE_BLOCK
    last_blk = (TABLE_ROWS - 1) // _DETILE_BLOCK
    spec = lambda a: pl.BlockSpec(
        (_DETILE_BLOCK, EMBED_DIM),
        lambda i, a=a: (jnp.minimum(a * nblk + i, last_blk), 0),
    )
    return pl.pallas_call(
        _detile_body,
        grid=(nblk,),
        in_specs=[spec(0), spec(1), spec(2), spec(3)],
        out_specs=pl.BlockSpec((_DETILE_BLOCK, LANES), lambda i: (i, 0)),
        out_shape=jax.ShapeDtypeStruct((WIDE_ROWS, LANES), jnp.float32),
    )(table, table, table, table)


_MLP_BLOCK = 2048


def _mlp_body(emb_ref, rmod_ref, w_ref, b_ref, g_ref, be_ref, out_ref):
    emb = emb_ref[...]
    group = lax.broadcasted_iota(jnp.int32, emb.shape, 1) // EMBED_DIM
    emb_sel = jnp.where(group == rmod_ref[...], emb, 0.0)
    h = lax.dot_general(
        emb_sel,
        w_ref[...],
        (((1,), (0,)), ((), ())),
        precision=lax.Precision.HIGHEST,
        preferred_element_type=jnp.float32,
    )
    h = h + b_ref[...]
    mu = jnp.mean(h, axis=1, keepdims=True)
    var = jnp.mean((h - mu) ** 2, axis=1, keepdims=True)
    hn = (h - mu) * lax.rsqrt(var + EPS)
    out_ref[...] = jnp.maximum(hn * g_ref[...] + be_ref[...], 0.0)


def _tc_mlp(emb, rmod, W_rep, b, gamma, beta):
    grid = (BATCH // _MLP_BLOCK,)
    return pl.pallas_call(
        _mlp_body,
        grid=grid,
        in_specs=[
            pl.BlockSpec((_MLP_BLOCK, LANES), lambda i: (i, 0)),
            pl.BlockSpec((_MLP_BLOCK, 1), lambda i: (i, 0)),
            pl.BlockSpec((LANES, HIDDEN_DIM), lambda i: (0, 0)),
            pl.BlockSpec((1, HIDDEN_DIM), lambda i: (0, 0)),
            pl.BlockSpec((1, HIDDEN_DIM), lambda i: (0, 0)),
            pl.BlockSpec((1, HIDDEN_DIM), lambda i: (0, 0)),
        ],
        out_specs=pl.BlockSpec((_MLP_BLOCK, HIDDEN_DIM), lambda i: (i, 0)),
        out_shape=jax.ShapeDtypeStruct((BATCH, HIDDEN_DIM), jnp.float32),
    )(emb, rmod, W_rep, b, gamma, beta)


def kernel(task_ids, table, W, b, gamma, beta):
    ids = task_ids.reshape(BATCH).astype(jnp.int32)
    lin = _tc_detile(table)
    idx2d = (ids % WIDE_ROWS).reshape(BATCH // GATHER_CHUNK, GATHER_CHUNK)
    rmod = (ids // WIDE_ROWS).reshape(BATCH, 1)
    emb128 = _sc_gather(lin, idx2d)
    W_rep = jnp.tile(W, (PACK, 1))
    return _tc_mlp(
        emb128,
        rmod,
        W_rep,
        b.reshape(1, HIDDEN_DIM),
        gamma.reshape(1, HIDDEN_DIM),
        beta.reshape(1, HIDDEN_DIM),
    )


# speedup vs baseline: 1.5465x; 1.0477x over previous
import functools

import jax
import jax.numpy as jnp
from jax import lax
from jax.experimental import pallas as pl
from jax.experimental.pallas import tpu as pltpu
from jax.experimental.pallas import tpu_sc as plsc

BATCH = 16384
EMBED_DIM = 32
HIDDEN_DIM = 64
EPS = 1e-5

LANES = 128
PACK = LANES // EMBED_DIM
TABLE_ROWS = 1000001
_DETILE_BLOCK = 8192
WIDE_ROWS = 31 * _DETILE_BLOCK

NUM_CORES = 2
NUM_SUBCORES = 16
NUM_WORKERS = NUM_CORES * NUM_SUBCORES
ROWS_PER_WORKER = BATCH // NUM_WORKERS
GATHER_CHUNK = 128
CHUNKS_PER_WORKER = ROWS_PER_WORKER // GATHER_CHUNK


def _sc_gather(lin, idx2d):
    mesh = plsc.VectorSubcoreMesh(core_axis_name="c", subcore_axis_name="s")

    @functools.partial(
        pl.kernel,
        mesh=mesh,
        out_type=jax.ShapeDtypeStruct((BATCH, LANES), jnp.float32),
        scratch_types=[
            pltpu.VMEM((CHUNKS_PER_WORKER, GATHER_CHUNK), jnp.int32),
            pltpu.VMEM((ROWS_PER_WORKER, LANES), jnp.float32),
            pltpu.SemaphoreType.DMA,
        ],
    )
    def gather_kernel(lin_hbm, idx_hbm, out_hbm, idx_v, rows_v, sem):
        wid = lax.axis_index("s") * NUM_CORES + lax.axis_index("c")
        pltpu.sync_copy(idx_hbm.at[pl.ds(wid * CHUNKS_PER_WORKER, CHUNKS_PER_WORKER)], idx_v)
        copies = []
        for j in range(CHUNKS_PER_WORKER):
            copies.append(
                pltpu.async_copy(
                    lin_hbm.at[idx_v.at[j]],
                    rows_v.at[pl.ds(j * GATHER_CHUNK, GATHER_CHUNK)],
                    sem,
                )
            )
        for c in copies:
            c.wait()
        pltpu.sync_copy(rows_v, out_hbm.at[pl.ds(wid * ROWS_PER_WORKER, ROWS_PER_WORKER)])

    return gather_kernel(lin, idx2d)


def _detile_body(in0, in1, in2, in3, out_ref):
    out_ref[:, 0:EMBED_DIM] = in0[...]
    out_ref[:, EMBED_DIM : 2 * EMBED_DIM] = in1[...]
    out_ref[:, 2 * EMBED_DIM : 3 * EMBED_DIM] = in2[...]
    out_ref[:, 3 * EMBED_DIM :] = in3[...]


def _tc_detile(table):
    nblk = WIDE_ROWS // _DETILE_BLOCK
    last_blk = (TABLE_ROWS - 1) // _DETILE_BLOCK
    spec = lambda a: pl.BlockSpec(
        (_DETILE_BLOCK, EMBED_DIM),
        lambda i, a=a: (jnp.minimum(a * nblk + i, last_blk), 0),
    )
    return pl.pallas_call(
        _detile_body,
        grid=(nblk,),
        in_specs=[spec(0), spec(1), spec(2), spec(3)],
        out_specs=pl.BlockSpec((_DETILE_BLOCK, LANES), lambda i: (i, 0)),
        out_shape=jax.ShapeDtypeStruct((WIDE_ROWS, LANES), jnp.float32),
    )(table, table, table, table)


_MLP_BLOCK = 2048


def _mlp_body(emb_ref, rmod_ref, w_ref, b_ref, g_ref, be_ref, out_ref):
    emb = emb_ref[...]
    group = lax.broadcasted_iota(jnp.int32, emb.shape, 1) // EMBED_DIM
    emb_sel = jnp.where(group == rmod_ref[...], emb, 0.0)
    h = lax.dot_general(
        emb_sel,
        w_ref[...],
        (((1,), (0,)), ((), ())),
        precision=lax.Precision.HIGHEST,
        preferred_element_type=jnp.float32,
    )
    h = h + b_ref[...]
    mu = jnp.mean(h, axis=1, keepdims=True)
    var = jnp.mean((h - mu) ** 2, axis=1, keepdims=True)
    hn = (h - mu) * lax.rsqrt(var + EPS)
    out_ref[...] = jnp.maximum(hn * g_ref[...] + be_ref[...], 0.0)


def _tc_mlp(emb, rmod, W_rep, b, gamma, beta):
    grid = (BATCH // _MLP_BLOCK,)
    return pl.pallas_call(
        _mlp_body,
        grid=grid,
        in_specs=[
            pl.BlockSpec((_MLP_BLOCK, LANES), lambda i: (i, 0)),
            pl.BlockSpec((_MLP_BLOCK, 1), lambda i: (i, 0)),
            pl.BlockSpec((LANES, HIDDEN_DIM), lambda i: (0, 0)),
            pl.BlockSpec((1, HIDDEN_DIM), lambda i: (0, 0)),
            pl.BlockSpec((1, HIDDEN_DIM), lambda i: (0, 0)),
            pl.BlockSpec((1, HIDDEN_DIM), lambda i: (0, 0)),
        ],
        out_specs=pl.BlockSpec((_MLP_BLOCK, HIDDEN_DIM), lambda i: (i, 0)),
        out_shape=jax.ShapeDtypeStruct((BATCH, HIDDEN_DIM), jnp.float32),
    )(emb, rmod, W_rep, b, gamma, beta)


def kernel(task_ids, table, W, b, gamma, beta):
    ids = task_ids.reshape(BATCH).astype(jnp.int32)
    lin = _tc_detile(table)
    idx2d = (ids % WIDE_ROWS).reshape(BATCH // GATHER_CHUNK, GATHER_CHUNK)
    rmod = (ids // WIDE_ROWS).reshape(BATCH, 1)
    emb128 = _sc_gather(lin, idx2d)
    W_rep = jnp.tile(W, (PACK, 1))
    return _tc_mlp(
        emb128,
        rmod,
        W_rep,
        b.reshape(1, HIDDEN_DIM),
        gamma.reshape(1, HIDDEN_DIM),
        beta.reshape(1, HIDDEN_DIM),
    )


# speedup vs baseline: 2.6076x; 1.6861x over previous
import functools

import jax
import jax.numpy as jnp
from jax import lax
from jax.experimental import pallas as pl
from jax.experimental.pallas import tpu as pltpu
from jax.experimental.pallas import tpu_sc as plsc

BATCH = 16384
EMBED_DIM = 32
HIDDEN_DIM = 64
EPS = 1e-5

NUM_CORES = 2
NUM_SUBCORES = 16
NUM_WORKERS = NUM_CORES * NUM_SUBCORES
ROWS_PER_WORKER = BATCH // NUM_WORKERS
GROUP = 16
NGROUPS = ROWS_PER_WORKER // GROUP
PRIME = 4


def _sc_gather(table, ids2d):
    mesh = plsc.VectorSubcoreMesh(core_axis_name="c", subcore_axis_name="s")

    @functools.partial(
        pl.kernel,
        mesh=mesh,
        out_type=jax.ShapeDtypeStruct((BATCH, EMBED_DIM), jnp.float32),
        scratch_types=[
            pltpu.VMEM((ROWS_PER_WORKER,), jnp.int32),
            pltpu.VMEM((ROWS_PER_WORKER, EMBED_DIM), jnp.float32),
            pltpu.SemaphoreType.DMA,
            pltpu.SemaphoreType.DMA,
        ],
    )
    def gather_kernel(table_hbm, idx_hbm, out_hbm, idx_v, rows_v, sem_i, sem):
        wid = lax.axis_index("s") * NUM_CORES + lax.axis_index("c")
        pltpu.async_copy(idx_hbm.at[wid], idx_v, sem_i).wait()

        def fire_group(base):
            v = idx_v[pl.ds(base, GROUP)]
            for j in range(GROUP):
                pltpu.async_copy(
                    table_hbm.at[pl.ds(v[j], 1)], rows_v.at[pl.ds(base + j, 1)], sem
                )

        def wait_group():
            for _ in range(GROUP):
                pltpu.make_async_copy(
                    table_hbm.at[pl.ds(0, 1)], rows_v.at[pl.ds(0, 1)], sem
                ).wait()

        for g in range(PRIME):
            fire_group(g * GROUP)

        @pl.loop(PRIME, NGROUPS)
        def _(g):
            fire_group(g * GROUP)
            wait_group()

        for _ in range(PRIME):
            wait_group()

        pltpu.sync_copy(rows_v, out_hbm.at[pl.ds(wid * ROWS_PER_WORKER, ROWS_PER_WORKER)])

    return gather_kernel(table, ids2d)


_MLP_BLOCK = 2048


def _mlp_body(emb_ref, w_ref, b_ref, g_ref, be_ref, out_ref):
    h = lax.dot_general(
        emb_ref[...],
        w_ref[...],
        (((1,), (0,)), ((), ())),
        precision=lax.Precision.HIGHEST,
        preferred_element_type=jnp.float32,
    )
    h = h + b_ref[...]
    mu = jnp.mean(h, axis=1, keepdims=True)
    var = jnp.mean((h - mu) ** 2, axis=1, keepdims=True)
    hn = (h - mu) * lax.rsqrt(var + EPS)
    out_ref[...] = jnp.maximum(hn * g_ref[...] + be_ref[...], 0.0)


def _tc_mlp(emb, W, b, gamma, beta):
    grid = (BATCH // _MLP_BLOCK,)
    return pl.pallas_call(
        _mlp_body,
        grid=grid,
        in_specs=[
            pl.BlockSpec((_MLP_BLOCK, EMBED_DIM), lambda i: (i, 0)),
            pl.BlockSpec((EMBED_DIM, HIDDEN_DIM), lambda i: (0, 0)),
            pl.BlockSpec((1, HIDDEN_DIM), lambda i: (0, 0)),
            pl.BlockSpec((1, HIDDEN_DIM), lambda i: (0, 0)),
            pl.BlockSpec((1, HIDDEN_DIM), lambda i: (0, 0)),
        ],
        out_specs=pl.BlockSpec((_MLP_BLOCK, HIDDEN_DIM), lambda i: (i, 0)),
        out_shape=jax.ShapeDtypeStruct((BATCH, HIDDEN_DIM), jnp.float32),
    )(emb, W, b, gamma, beta)


def kernel(task_ids, table, W, b, gamma, beta):
    ids2d = task_ids.reshape(NUM_WORKERS, ROWS_PER_WORKER).astype(jnp.int32)
    emb = _sc_gather(table, ids2d)
    return _tc_mlp(
        emb,
        W,
        b.reshape(1, HIDDEN_DIM),
        gamma.reshape(1, HIDDEN_DIM),
        beta.reshape(1, HIDDEN_DIM),
    )
